# baseline (speedup 1.0000x reference)
import functools

import jax
import jax.numpy as jnp
from jax import lax
from jax.experimental import pallas as pl
from jax.experimental.pallas import tpu as pltpu
from jax.experimental.pallas import tpu_sc as plsc

B = 16384
F = 16
NC = 2
NS = 16
NW = NC * NS
BPW = B // NW


def _sc_body(user_hbm, item_hbm, bi_hbm, gu_hbm, gi_hbm, dgu_hbm, dgi_hbm,
             xui_out, beta_out, guo_out, gio_out,
             uidx_v, iidx_v, gu_v, gi_v, dgu_v, dgi_v, bi_v, xui_v,
             sem_u, sem_i, sem_du, sem_di, sem_b):
    wid = lax.axis_index("s") * NC + lax.axis_index("c")
    base = wid * BPW

    pltpu.sync_copy(user_hbm.at[pl.ds(base, BPW)], uidx_v)
    pltpu.sync_copy(item_hbm.at[pl.ds(base, BPW)], iidx_v)

    cu = pltpu.async_copy(gu_hbm.at[uidx_v], gu_v, sem_u)
    ci = pltpu.async_copy(gi_hbm.at[iidx_v], gi_v, sem_i)
    cdu = pltpu.async_copy(dgu_hbm.at[uidx_v], dgu_v, sem_du)
    cdi = pltpu.async_copy(dgi_hbm.at[iidx_v], dgi_v, sem_di)
    cb = pltpu.async_copy(bi_hbm.at[iidx_v], bi_v, sem_b)
    cu.wait()
    cdu.wait()
    ci.wait()
    cdi.wait()
    cb.wait()

    lane = lax.iota(jnp.int32, F)

    def grp(g, carry):
        b16 = g * F
        bvec = bi_v[pl.ds(b16, F)]
        acc = jnp.zeros((F,), jnp.float32)
        for r in range(F):
            gu_row = gu_v[b16 + r, :] + dgu_v[b16 + r, :]
            gi_row = gi_v[b16 + r, :] + dgi_v[b16 + r, :]
            gu_v[b16 + r, :] = gu_row
            gi_v[b16 + r, :] = gi_row
            s = jnp.sum(gu_row * gi_row)
            acc = jnp.where(lane == r, s, acc)
        xui_v[pl.ds(b16, F)] = acc + bvec
        return carry

    lax.fori_loop(0, BPW // F, grp, 0)

    pltpu.sync_copy(gu_v, guo_out.at[pl.ds(base, BPW)])
    pltpu.sync_copy(gi_v, gio_out.at[pl.ds(base, BPW)])
    pltpu.sync_copy(xui_v, xui_out.at[pl.ds(base, BPW)])
    pltpu.sync_copy(bi_v, beta_out.at[pl.ds(base, BPW)])


@jax.jit
def _run(user, item, Bi, Gu, Gi, Delta_Gu, Delta_Gi):
    f = pl.kernel(
        _sc_body,
        out_type=(
            jax.ShapeDtypeStruct((B,), jnp.float32),
            jax.ShapeDtypeStruct((B,), jnp.float32),
            jax.ShapeDtypeStruct((B, F), jnp.float32),
            jax.ShapeDtypeStruct((B, F), jnp.float32),
        ),
        mesh=plsc.VectorSubcoreMesh(core_axis_name="c", subcore_axis_name="s"),
        compiler_params=pltpu.CompilerParams(
            needs_layout_passes=False, use_tc_tiling_on_sc=False),
        scratch_types=[
            pltpu.VMEM((BPW,), jnp.int32),
            pltpu.VMEM((BPW,), jnp.int32),
            pltpu.VMEM((BPW, F), jnp.float32),
            pltpu.VMEM((BPW, F), jnp.float32),
            pltpu.VMEM((BPW, F), jnp.float32),
            pltpu.VMEM((BPW, F), jnp.float32),
            pltpu.VMEM((BPW,), jnp.float32),
            pltpu.VMEM((BPW,), jnp.float32),
            pltpu.SemaphoreType.DMA,
            pltpu.SemaphoreType.DMA,
            pltpu.SemaphoreType.DMA,
            pltpu.SemaphoreType.DMA,
            pltpu.SemaphoreType.DMA,
        ],
    )
    return f(user, item, Bi, Gu, Gi, Delta_Gu, Delta_Gi)


def kernel(user, item, Bi, Gu, Gi, Delta_Gu, Delta_Gi):
    xui, beta_i, gamma_u, gamma_i = _run(
        user, item, Bi, Gu, Gi, Delta_Gu, Delta_Gi)
    return (xui, beta_i, gamma_u, gamma_i)


# speedup vs baseline: 1.3347x; 1.3347x over previous
import functools

import jax
import jax.numpy as jnp
from jax import lax
from jax.experimental import pallas as pl
from jax.experimental.pallas import tpu as pltpu
from jax.experimental.pallas import tpu_sc as plsc

B = 16384
F = 16
NC = 2
NS = 16
NW = NC * NS
BPW = B // NW
V = 1000000
TW = 8192


def _tc_body(gt_ref, dgt_ref, out_ref):
    out_ref[...] = jnp.transpose(gt_ref[...] + dgt_ref[...], (1, 0))


def _sum_rowmajor(table_t, delta_t):
    grid = (V + TW - 1) // TW
    return pl.pallas_call(
        _tc_body,
        grid=(grid,),
        in_specs=[
            pl.BlockSpec((F, TW), lambda i: (0, i)),
            pl.BlockSpec((F, TW), lambda i: (0, i)),
        ],
        out_specs=pl.BlockSpec((TW, F), lambda i: (i, 0)),
        out_shape=jax.ShapeDtypeStruct((V, F), jnp.float32),
    )(table_t, delta_t)


def _sc_body(user_hbm, item_hbm, bi_hbm, su_hbm, si_hbm,
             xui_out, beta_out, guo_out, gio_out,
             uidx_v, iidx_v, gu_v, gi_v, bi_v, xui_v,
             sem_u, sem_i, sem_b):
    wid = lax.axis_index("s") * NC + lax.axis_index("c")
    base = wid * BPW

    pltpu.sync_copy(user_hbm.at[pl.ds(base, BPW)], uidx_v)
    pltpu.sync_copy(item_hbm.at[pl.ds(base, BPW)], iidx_v)

    cu = pltpu.async_copy(su_hbm.at[uidx_v], gu_v, sem_u)
    ci = pltpu.async_copy(si_hbm.at[iidx_v], gi_v, sem_i)
    cb = pltpu.async_copy(bi_hbm.at[iidx_v], bi_v, sem_b)
    cu.wait()
    ci.wait()
    cb.wait()

    lane = lax.iota(jnp.int32, F)

    def grp(g, carry):
        b16 = g * F
        bvec = bi_v[pl.ds(b16, F)]
        acc = jnp.zeros((F,), jnp.float32)
        for r in range(F):
            s = jnp.sum(gu_v[b16 + r, :] * gi_v[b16 + r, :])
            acc = jnp.where(lane == r, s, acc)
        xui_v[pl.ds(b16, F)] = acc + bvec
        return carry

    lax.fori_loop(0, BPW // F, grp, 0)

    pltpu.sync_copy(gu_v, guo_out.at[pl.ds(base, BPW)])
    pltpu.sync_copy(gi_v, gio_out.at[pl.ds(base, BPW)])
    pltpu.sync_copy(xui_v, xui_out.at[pl.ds(base, BPW)])
    pltpu.sync_copy(bi_v, beta_out.at[pl.ds(base, BPW)])


def _gather_dot(user, item, Bi, Su, Si):
    f = pl.kernel(
        _sc_body,
        out_type=(
            jax.ShapeDtypeStruct((B,), jnp.float32),
            jax.ShapeDtypeStruct((B,), jnp.float32),
            jax.ShapeDtypeStruct((B, F), jnp.float32),
            jax.ShapeDtypeStruct((B, F), jnp.float32),
        ),
        mesh=plsc.VectorSubcoreMesh(core_axis_name="c", subcore_axis_name="s"),
        compiler_params=pltpu.CompilerParams(
            needs_layout_passes=False, use_tc_tiling_on_sc=False),
        scratch_types=[
            pltpu.VMEM((BPW,), jnp.int32),
            pltpu.VMEM((BPW,), jnp.int32),
            pltpu.VMEM((BPW, F), jnp.float32),
            pltpu.VMEM((BPW, F), jnp.float32),
            pltpu.VMEM((BPW,), jnp.float32),
            pltpu.VMEM((BPW,), jnp.float32),
            pltpu.SemaphoreType.DMA,
            pltpu.SemaphoreType.DMA,
            pltpu.SemaphoreType.DMA,
        ],
    )
    return f(user, item, Bi, Su, Si)


@jax.jit
def _run(user, item, Bi, GuT, GiT, Delta_GuT, Delta_GiT):
    su = _sum_rowmajor(GuT, Delta_GuT)
    si = _sum_rowmajor(GiT, Delta_GiT)
    return _gather_dot(user, item, Bi, su, si)


def kernel(user, item, Bi, Gu, Gi, Delta_Gu, Delta_Gi):
    return _run(user, item, Bi, Gu.T, Gi.T, Delta_Gu.T, Delta_Gi.T)


# speedup vs baseline: 1.4230x; 1.0661x over previous
import functools

import jax
import jax.numpy as jnp
from jax import lax
from jax.experimental import pallas as pl
from jax.experimental.pallas import tpu as pltpu
from jax.experimental.pallas import tpu_sc as plsc

B = 16384
F = 16
NC = 2
NS = 16
NW = NC * NS
BPW = B // NW
V = 1000000
TW = 32768


def _tc_body(gt_ref, dgt_ref, out_ref):
    s = gt_ref[...] + dgt_ref[...]
    eye = jax.lax.broadcasted_iota(jnp.int32, (F, F), 0)
    eye = jnp.where(eye == jax.lax.broadcasted_iota(jnp.int32, (F, F), 1),
                    jnp.float32(1), jnp.float32(0))
    out_ref[...] = jax.lax.dot_general(
        s, eye, (((0,), (0,)), ((), ())),
        preferred_element_type=jnp.float32)


def _sum_rowmajor(table_t, delta_t):
    grid = (V + TW - 1) // TW
    return pl.pallas_call(
        _tc_body,
        grid=(grid,),
        in_specs=[
            pl.BlockSpec((F, TW), lambda i: (0, i)),
            pl.BlockSpec((F, TW), lambda i: (0, i)),
        ],
        out_specs=pl.BlockSpec((TW, F), lambda i: (i, 0)),
        out_shape=jax.ShapeDtypeStruct((V, F), jnp.float32),
        compiler_params=pltpu.CompilerParams(
            fuse_transposed_lhs_in_matmul=True),
    )(table_t, delta_t)


def _sc_body(user_hbm, item_hbm, bi_hbm, su_hbm, si_hbm,
             xui_out, beta_out, guo_out, gio_out,
             uidx_v, iidx_v, gu_v, gi_v, bi_v, xui_v,
             sem_u, sem_i, sem_b):
    wid = lax.axis_index("s") * NC + lax.axis_index("c")
    base = wid * BPW

    pltpu.sync_copy(user_hbm.at[pl.ds(base, BPW)], uidx_v)
    pltpu.sync_copy(item_hbm.at[pl.ds(base, BPW)], iidx_v)

    cu = pltpu.async_copy(su_hbm.at[uidx_v], gu_v, sem_u)
    ci = pltpu.async_copy(si_hbm.at[iidx_v], gi_v, sem_i)
    cb = pltpu.async_copy(bi_hbm.at[iidx_v], bi_v, sem_b)
    cu.wait()
    ci.wait()
    cb.wait()

    lane = lax.iota(jnp.int32, F)

    def grp(g, carry):
        b16 = g * F
        bvec = bi_v[pl.ds(b16, F)]
        acc = jnp.zeros((F,), jnp.float32)
        for r in range(F):
            s = jnp.sum(gu_v[b16 + r, :] * gi_v[b16 + r, :])
            acc = jnp.where(lane == r, s, acc)
        xui_v[pl.ds(b16, F)] = acc + bvec
        return carry

    lax.fori_loop(0, BPW // F, grp, 0)

    pltpu.sync_copy(gu_v, guo_out.at[pl.ds(base, BPW)])
    pltpu.sync_copy(gi_v, gio_out.at[pl.ds(base, BPW)])
    pltpu.sync_copy(xui_v, xui_out.at[pl.ds(base, BPW)])
    pltpu.sync_copy(bi_v, beta_out.at[pl.ds(base, BPW)])


def _gather_dot(user, item, Bi, Su, Si):
    f = pl.kernel(
        _sc_body,
        out_type=(
            jax.ShapeDtypeStruct((B,), jnp.float32),
            jax.ShapeDtypeStruct((B,), jnp.float32),
            jax.ShapeDtypeStruct((B, F), jnp.float32),
            jax.ShapeDtypeStruct((B, F), jnp.float32),
        ),
        mesh=plsc.VectorSubcoreMesh(core_axis_name="c", subcore_axis_name="s"),
        compiler_params=pltpu.CompilerParams(
            needs_layout_passes=False, use_tc_tiling_on_sc=False),
        scratch_types=[
            pltpu.VMEM((BPW,), jnp.int32),
            pltpu.VMEM((BPW,), jnp.int32),
            pltpu.VMEM((BPW, F), jnp.float32),
            pltpu.VMEM((BPW, F), jnp.float32),
            pltpu.VMEM((BPW,), jnp.float32),
            pltpu.VMEM((BPW,), jnp.float32),
            pltpu.SemaphoreType.DMA,
            pltpu.SemaphoreType.DMA,
            pltpu.SemaphoreType.DMA,
        ],
    )
    return f(user, item, Bi, Su, Si)


@jax.jit
def _run(user, item, Bi, GuT, GiT, Delta_GuT, Delta_GiT):
    su = _sum_rowmajor(GuT, Delta_GuT)
    si = _sum_rowmajor(GiT, Delta_GiT)
    return _gather_dot(user, item, Bi, su, si)


def kernel(user, item, Bi, Gu, Gi, Delta_Gu, Delta_Gi):
    return _run(user, item, Bi, Gu.T, Gi.T, Delta_Gu.T, Delta_Gi.T)


# speedup vs baseline: 1.4260x; 1.0021x over previous
import functools

import jax
import jax.numpy as jnp
from jax import lax
from jax.experimental import pallas as pl
from jax.experimental.pallas import tpu as pltpu
from jax.experimental.pallas import tpu_sc as plsc

B = 16384
F = 16
NC = 2
NS = 16
NW = NC * NS
BPW = B // NW
V = 1000000
TW = 32768


def _tc_body(gt_ref, dgt_ref, out_ref):
    s = gt_ref[...] + dgt_ref[...]
    eye = jax.lax.broadcasted_iota(jnp.int32, (F, F), 0)
    eye = jnp.where(eye == jax.lax.broadcasted_iota(jnp.int32, (F, F), 1),
                    jnp.float32(1), jnp.float32(0))
    t = jax.lax.dot_general(
        s, eye, (((0,), (0,)), ((), ())),
        preferred_element_type=jnp.float32)
    out_ref[...] = t.reshape(TW // 8, 8, F)


def _sum_rowmajor(table_t, delta_t):
    grid = (V + TW - 1) // TW
    return pl.pallas_call(
        _tc_body,
        grid=(grid,),
        in_specs=[
            pl.BlockSpec((F, TW), lambda i: (0, i)),
            pl.BlockSpec((F, TW), lambda i: (0, i)),
        ],
        out_specs=pl.BlockSpec((TW // 8, 8, F), lambda i: (i, 0, 0)),
        out_shape=jax.ShapeDtypeStruct((V // 8, 8, F), jnp.float32),
        compiler_params=pltpu.CompilerParams(
            fuse_transposed_lhs_in_matmul=True),
    )(table_t, delta_t)


def _sc_body(user_hbm, item_hbm, bi_hbm, su_hbm, si_hbm,
             xui_out, beta_out, guo_out, gio_out,
             uidx_v, iidx_v, gu_v, gi_v, bi_v, xui_v,
             sem_u, sem_i, sem_b):
    wid = lax.axis_index("s") * NC + lax.axis_index("c")
    base = wid * BPW

    pltpu.sync_copy(user_hbm.at[pl.ds(base, BPW)], uidx_v)
    pltpu.sync_copy(item_hbm.at[pl.ds(base, BPW)], iidx_v)

    cu = pltpu.async_copy(su_hbm.at[uidx_v], gu_v, sem_u)
    ci = pltpu.async_copy(si_hbm.at[iidx_v], gi_v, sem_i)
    cb = pltpu.async_copy(bi_hbm.at[iidx_v], bi_v, sem_b)
    cu.wait()
    ci.wait()
    cb.wait()

    lane = lax.iota(jnp.int32, F)

    def grp(g, carry):
        b16 = g * F
        bvec = bi_v[pl.ds(b16, F)]
        acc = jnp.zeros((F,), jnp.float32)
        for r in range(F):
            s = jnp.sum(gu_v[b16 + r, :] * gi_v[b16 + r, :])
            acc = jnp.where(lane == r, s, acc)
        xui_v[pl.ds(b16, F)] = acc + bvec
        return carry

    lax.fori_loop(0, BPW // F, grp, 0)

    pltpu.sync_copy(gu_v, guo_out.at[pl.ds(base, BPW)])
    pltpu.sync_copy(gi_v, gio_out.at[pl.ds(base, BPW)])
    pltpu.sync_copy(xui_v, xui_out.at[pl.ds(base, BPW)])
    pltpu.sync_copy(bi_v, beta_out.at[pl.ds(base, BPW)])


def _gather_dot(user, item, Bi, Su, Si):
    f = pl.kernel(
        _sc_body,
        out_type=(
            jax.ShapeDtypeStruct((B,), jnp.float32),
            jax.ShapeDtypeStruct((B,), jnp.float32),
            jax.ShapeDtypeStruct((B, F), jnp.float32),
            jax.ShapeDtypeStruct((B, F), jnp.float32),
        ),
        mesh=plsc.VectorSubcoreMesh(core_axis_name="c", subcore_axis_name="s"),
        compiler_params=pltpu.CompilerParams(
            needs_layout_passes=False, use_tc_tiling_on_sc=False),
        scratch_types=[
            pltpu.VMEM((BPW,), jnp.int32),
            pltpu.VMEM((BPW,), jnp.int32),
            pltpu.VMEM((BPW, F), jnp.float32),
            pltpu.VMEM((BPW, F), jnp.float32),
            pltpu.VMEM((BPW,), jnp.float32),
            pltpu.VMEM((BPW,), jnp.float32),
            pltpu.SemaphoreType.DMA,
            pltpu.SemaphoreType.DMA,
            pltpu.SemaphoreType.DMA,
        ],
    )
    return f(user, item, Bi, Su, Si)


@jax.jit
def _run(user, item, Bi, GuT, GiT, Delta_GuT, Delta_GiT):
    su = _sum_rowmajor(GuT, Delta_GuT).reshape(V, F)
    si = _sum_rowmajor(GiT, Delta_GiT).reshape(V, F)
    return _gather_dot(user, item, Bi, su, si)


def kernel(user, item, Bi, Gu, Gi, Delta_Gu, Delta_Gi):
    return _run(user, item, Bi, Gu.T, Gi.T, Delta_Gu.T, Delta_Gi.T)


# speedup vs baseline: 3.3102x; 2.3213x over previous
import jax
import jax.numpy as jnp
from jax import lax
from jax.experimental import pallas as pl
from jax.experimental.pallas import tpu as pltpu
from jax.experimental.pallas import tpu_sc as plsc

B = 16384
F = 16
NC = 2
NS = 16
NW = NC * NS
BPW = B // NW
Q = 128
V = 1000000
TW = 32768
GR = (V + TW - 1) // TW


def _eye(rows):
    r = lax.broadcasted_iota(jnp.int32, (rows, 128), 0)
    c = lax.broadcasted_iota(jnp.int32, (rows, 128), 1)
    return jnp.where(r == c, jnp.float32(1), jnp.float32(0))


def _tc_u_body(gt_ref, dgt_ref, out_ref):
    s = gt_ref[...] + dgt_ref[...]
    out_ref[...] = lax.dot_general(
        s, _eye(F), (((0,), (0,)), ((), ())),
        preferred_element_type=jnp.float32)


def _tc_i_body(gt_ref, dgt_ref, bi_ref, out_ref):
    s = jnp.concatenate(
        [gt_ref[...] + dgt_ref[...], bi_ref[...].reshape(1, TW)], axis=0)
    out_ref[...] = lax.dot_general(
        s, _eye(F + 1), (((0,), (0,)), ((), ())),
        preferred_element_type=jnp.float32)


def _sum_padded_u(table_t, delta_t):
    return pl.pallas_call(
        _tc_u_body,
        grid=(GR,),
        in_specs=[
            pl.BlockSpec((F, TW), lambda i: (0, i)),
            pl.BlockSpec((F, TW), lambda i: (0, i)),
        ],
        out_specs=pl.BlockSpec((TW, 128), lambda i: (i, 0)),
        out_shape=jax.ShapeDtypeStruct((V, 128), jnp.float32),
        compiler_params=pltpu.CompilerParams(
            fuse_transposed_lhs_in_matmul=True),
    )(table_t, delta_t)


def _sum_padded_i(table_t, delta_t, bi):
    return pl.pallas_call(
        _tc_i_body,
        grid=(GR,),
        in_specs=[
            pl.BlockSpec((F, TW), lambda i: (0, i)),
            pl.BlockSpec((F, TW), lambda i: (0, i)),
            pl.BlockSpec((TW,), lambda i: (i,)),
        ],
        out_specs=pl.BlockSpec((TW, 128), lambda i: (i, 0)),
        out_shape=jax.ShapeDtypeStruct((V, 128), jnp.float32),
        compiler_params=pltpu.CompilerParams(
            fuse_transposed_lhs_in_matmul=True),
    )(table_t, delta_t, bi)


def _sc_body(user_hbm, item_hbm, su_hbm, si_hbm,
             xui_out, beta_out, guo_out, gio_out,
             uidx_v, iidx_v, bufu, bufi, gu_st, gi_st, xui_v, beta_v,
             sem_u, sem_i):
    wid = lax.axis_index("s") * NC + lax.axis_index("c")
    base = wid * BPW
    lane = lax.iota(jnp.int32, F)

    for p in range(BPW // Q):
        pltpu.sync_copy(user_hbm.at[pl.ds(base + p * Q, Q)], uidx_v.at[p])
        pltpu.sync_copy(item_hbm.at[pl.ds(base + p * Q, Q)], iidx_v.at[p])

    for p in range(BPW // Q):
        cu = pltpu.async_copy(su_hbm.at[uidx_v.at[p]], bufu, sem_u)
        ci = pltpu.async_copy(si_hbm.at[iidx_v.at[p]], bufi, sem_i)
        cu.wait()
        ci.wait()

        def grp(g, carry, p=p):
            acc = jnp.zeros((F,), jnp.float32)
            bacc = jnp.zeros((F,), jnp.float32)
            for r in range(F):
                row = g * F + r
                u_vec = bufu[row, pl.ds(0, F)]
                i_vec = bufi[row, pl.ds(0, F)]
                b16 = bufi[row, pl.ds(F, F)]
                b = b16[0]
                s = jnp.sum(u_vec * i_vec)
                acc = jnp.where(lane == r, s + b, acc)
                bacc = jnp.where(lane == r, b, bacc)
                st_row = p * 16 + 2 * g + (r // 8)
                gu_st[st_row, pl.ds((r % 8) * F, F)] = u_vec
                gi_st[st_row, pl.ds((r % 8) * F, F)] = i_vec
            xui_v[pl.ds(p * Q + g * F, F)] = acc
            beta_v[pl.ds(p * Q + g * F, F)] = bacc
            return carry

        lax.fori_loop(0, Q // F, grp, 0)

    pltpu.sync_copy(gu_st, guo_out.at[pl.ds(wid * 64, 64), :])
    pltpu.sync_copy(gi_st, gio_out.at[pl.ds(wid * 64, 64), :])
    pltpu.sync_copy(xui_v, xui_out.at[pl.ds(base, BPW)])
    pltpu.sync_copy(beta_v, beta_out.at[pl.ds(base, BPW)])


def _gather_dot(user, item, Su, Si):
    f = pl.kernel(
        _sc_body,
        out_type=(
            jax.ShapeDtypeStruct((B,), jnp.float32),
            jax.ShapeDtypeStruct((B,), jnp.float32),
            jax.ShapeDtypeStruct((B * F // 128, 128), jnp.float32),
            jax.ShapeDtypeStruct((B * F // 128, 128), jnp.float32),
        ),
        mesh=plsc.VectorSubcoreMesh(core_axis_name="c", subcore_axis_name="s"),
        compiler_params=pltpu.CompilerParams(needs_layout_passes=False),
        scratch_types=[
            pltpu.VMEM((BPW // Q, Q), jnp.int32),
            pltpu.VMEM((BPW // Q, Q), jnp.int32),
            pltpu.VMEM((Q, 128), jnp.float32),
            pltpu.VMEM((Q, 128), jnp.float32),
            pltpu.VMEM((64, 128), jnp.float32),
            pltpu.VMEM((64, 128), jnp.float32),
            pltpu.VMEM((BPW,), jnp.float32),
            pltpu.VMEM((BPW,), jnp.float32),
            pltpu.SemaphoreType.DMA,
            pltpu.SemaphoreType.DMA,
        ],
    )
    return f(user, item, Su, Si)


@jax.jit
def _run(user, item, Bi, GuT, GiT, Delta_GuT, Delta_GiT):
    su = _sum_padded_u(GuT, Delta_GuT)
    si = _sum_padded_i(GiT, Delta_GiT, Bi)
    xui, beta_i, guo, gio = _gather_dot(user, item, su, si)
    return xui, beta_i, guo.reshape(B, F), gio.reshape(B, F)


def kernel(user, item, Bi, Gu, Gi, Delta_Gu, Delta_Gi):
    return _run(user, item, Bi, Gu.T, Gi.T, Delta_Gu.T, Delta_Gi.T)


# speedup vs baseline: 5.0414x; 1.5230x over previous
import jax
import jax.numpy as jnp
from jax import lax
from jax.experimental import pallas as pl
from jax.experimental.pallas import tpu as pltpu
from jax.experimental.pallas import tpu_sc as plsc

B = 16384
F = 16
NC = 2
NS = 16
NW = NC * NS
BPW = B // NW
Q = 128
V = 1000000
TW = 32768
GR = (V + TW - 1) // TW


def _eye(rows):
    r = lax.broadcasted_iota(jnp.int32, (rows, 128), 0)
    c = lax.broadcasted_iota(jnp.int32, (rows, 128), 1)
    return jnp.where(r == c, jnp.float32(1), jnp.float32(0))


def _tc_body(gut_ref, dgut_ref, git_ref, dgit_ref, bi_ref, out_ref):
    s = jnp.concatenate(
        [gut_ref[...] + dgut_ref[...],
         git_ref[...] + dgit_ref[...],
         bi_ref[...].reshape(1, TW)], axis=0)
    out_ref[...] = lax.dot_general(
        s, _eye(2 * F + 1), (((0,), (0,)), ((), ())),
        preferred_element_type=jnp.float32)


def _sum_padded(gut, dgut, git, dgit, bi):
    return pl.pallas_call(
        _tc_body,
        grid=(GR,),
        in_specs=[
            pl.BlockSpec((F, TW), lambda i: (0, i)),
            pl.BlockSpec((F, TW), lambda i: (0, i)),
            pl.BlockSpec((F, TW), lambda i: (0, i)),
            pl.BlockSpec((F, TW), lambda i: (0, i)),
            pl.BlockSpec((TW,), lambda i: (i,)),
        ],
        out_specs=pl.BlockSpec((TW, 128), lambda i: (i, 0)),
        out_shape=jax.ShapeDtypeStruct((V, 128), jnp.float32),
        compiler_params=pltpu.CompilerParams(
            fuse_transposed_lhs_in_matmul=True),
    )(gut, dgut, git, dgit, bi)


def _sc_body(user_hbm, item_hbm, s_hbm,
             xui_out, beta_out, guo_out, gio_out,
             uidx_v, iidx_v, bufu, bufi, gu_st, gi_st, xui_v, beta_v,
             sem_u, sem_i):
    wid = lax.axis_index("s") * NC + lax.axis_index("c")
    base = wid * BPW
    lane = lax.iota(jnp.int32, F)

    for p in range(BPW // Q):
        pltpu.sync_copy(user_hbm.at[pl.ds(base + p * Q, Q)], uidx_v.at[p])
        pltpu.sync_copy(item_hbm.at[pl.ds(base + p * Q, Q)], iidx_v.at[p])

    for p in range(BPW // Q):
        cu = pltpu.async_copy(s_hbm.at[uidx_v.at[p]], bufu, sem_u)
        ci = pltpu.async_copy(s_hbm.at[iidx_v.at[p]], bufi, sem_i)
        cu.wait()
        ci.wait()

        def grp(g, carry, p=p):
            acc = jnp.zeros((F,), jnp.float32)
            bacc = jnp.zeros((F,), jnp.float32)
            for r in range(F):
                row = g * F + r
                u_vec = bufu[row, pl.ds(0, F)]
                i_vec = bufi[row, pl.ds(F, F)]
                b16 = bufi[row, pl.ds(2 * F, F)]
                b = b16[0]
                s = jnp.sum(u_vec * i_vec)
                acc = jnp.where(lane == r, s + b, acc)
                bacc = jnp.where(lane == r, b, bacc)
                st_row = p * 16 + 2 * g + (r // 8)
                gu_st[st_row, pl.ds((r % 8) * F, F)] = u_vec
                gi_st[st_row, pl.ds((r % 8) * F, F)] = i_vec
            xui_v[pl.ds(p * Q + g * F, F)] = acc
            beta_v[pl.ds(p * Q + g * F, F)] = bacc
            return carry

        lax.fori_loop(0, Q // F, grp, 0)

    pltpu.sync_copy(gu_st, guo_out.at[pl.ds(wid * 64, 64), :])
    pltpu.sync_copy(gi_st, gio_out.at[pl.ds(wid * 64, 64), :])
    pltpu.sync_copy(xui_v, xui_out.at[pl.ds(base, BPW)])
    pltpu.sync_copy(beta_v, beta_out.at[pl.ds(base, BPW)])


def _gather_dot(user, item, S):
    f = pl.kernel(
        _sc_body,
        out_type=(
            jax.ShapeDtypeStruct((B,), jnp.float32),
            jax.ShapeDtypeStruct((B,), jnp.float32),
            jax.ShapeDtypeStruct((B * F // 128, 128), jnp.float32),
            jax.ShapeDtypeStruct((B * F // 128, 128), jnp.float32),
        ),
        mesh=plsc.VectorSubcoreMesh(core_axis_name="c", subcore_axis_name="s"),
        compiler_params=pltpu.CompilerParams(needs_layout_passes=False),
        scratch_types=[
            pltpu.VMEM((BPW // Q, Q), jnp.int32),
            pltpu.VMEM((BPW // Q, Q), jnp.int32),
            pltpu.VMEM((Q, 128), jnp.float32),
            pltpu.VMEM((Q, 128), jnp.float32),
            pltpu.VMEM((64, 128), jnp.float32),
            pltpu.VMEM((64, 128), jnp.float32),
            pltpu.VMEM((BPW,), jnp.float32),
            pltpu.VMEM((BPW,), jnp.float32),
            pltpu.SemaphoreType.DMA,
            pltpu.SemaphoreType.DMA,
        ],
    )
    return f(user, item, S)


@jax.jit
def _run(user, item, Bi, GuT, GiT, Delta_GuT, Delta_GiT):
    s = _sum_padded(GuT, Delta_GuT, GiT, Delta_GiT, Bi)
    xui, beta_i, guo, gio = _gather_dot(user, item, s)
    return xui, beta_i, guo.reshape(B, F), gio.reshape(B, F)


def kernel(user, item, Bi, Gu, Gi, Delta_Gu, Delta_Gi):
    return _run(user, item, Bi, Gu.T, Gi.T, Delta_Gu.T, Delta_Gi.T)
